# baseline (speedup 1.0000x reference)
import functools

import jax
import jax.numpy as jnp
from jax import lax
from jax.experimental import pallas as pl
from jax.experimental.pallas import tpu as pltpu
from jax.experimental.pallas import tpu_sc as plsc

MAZE = 32
P = MAZE * MAZE
D = 64
V = 4
PP = P // 2
TBL = V * V * PP
NC, NS, L = 2, 16, 16
NW = NC * NS
CH = 128
CPM = P // CH
CHP = CH // 2


def _sc_encode(maze_grid, cellcat, pos128, batch):
    nb = batch // NW
    rows_per_sub = TBL // NS

    mesh = plsc.VectorSubcoreMesh(core_axis_name="c", subcore_axis_name="s")

    @functools.partial(
        pl.kernel,
        out_type=jax.ShapeDtypeStruct((batch, P, D), jnp.float32),
        mesh=mesh,
        compiler_params=pltpu.CompilerParams(needs_layout_passes=False),
        scratch_types=[
            pltpu.VMEM_SHARED((TBL, 2 * D), jnp.float32),
            pltpu.VMEM((2 * D,), jnp.float32),
        ] + [pltpu.VMEM((MAZE, MAZE), jnp.int32) for _ in range(2)]
          + [pltpu.VMEM((CHP,), jnp.int32) for _ in range(2)]
          + [pltpu.VMEM((CHP, 2 * D), jnp.float32) for _ in range(2)]
          + [pltpu.VMEM((CH, D), jnp.float32) for _ in range(2)]
          + [pltpu.SemaphoreType.DMA for _ in range(6)],
    )
    def k(maze_hbm, cell_hbm, pos_hbm, out_hbm, tbl_sh, ccbuf, *ring):
        mbufs = ring[0:2]
        ibufs = ring[2:4]
        cbufs = ring[4:6]
        obufs = ring[6:8]
        msems = ring[8:10]
        gsems = ring[10:12]
        osems = ring[12:14]
        cid = lax.axis_index("c")
        sid = lax.axis_index("s")
        wid = sid * NC + cid
        b0 = wid * nb
        lanes = lax.iota(jnp.int32, L)

        pltpu.sync_copy(cell_hbm.at[sid], ccbuf)
        ccs = [ccbuf[pl.ds(j * L, L)] for j in range(2 * D // L)]
        row0 = sid * rows_per_sub
        for kk in range(rows_per_sub // CHP):
            bb = cbufs[kk % 2]
            pltpu.sync_copy(pos_hbm.at[pl.ds(kk * CHP, CHP)], bb)

            def add_row(r, _, bb=bb):
                for j in range(2 * D // L):
                    bb[r, pl.ds(j * L, L)] += ccs[j]
                return _

            lax.fori_loop(0, CHP, add_row, 0)
            pltpu.sync_copy(bb, tbl_sh.at[pl.ds(row0 + kk * CHP, CHP)])
        plsc.subcore_barrier()

        def issue_maze(m, mb):
            pltpu.async_copy(maze_hbm.at[b0 + m], mbufs[mb], msems[mb])

        def wait_maze(m, mb):
            pltpu.make_async_copy(maze_hbm.at[b0 + m], mbufs[mb],
                                  msems[mb]).wait()

        def build_and_gather(m, q, s, mb):
            mbuf = mbufs[mb]
            for j in range(CH // MAZE):
                rsel = lanes * 0 + (q * (CH // MAZE) + j)
                e = plsc.load_gather(mbuf, [rsel, 2 * lanes])
                o = plsc.load_gather(mbuf, [rsel, 2 * lanes + 1])
                ppb = q * CHP + j * L
                ibufs[s][pl.ds(j * L, L)] = (e * V + o) * PP + ppb + lanes
            pltpu.async_copy(tbl_sh.at[ibufs[s]], cbufs[s], gsems[s])

        def wait_gather(s):
            pltpu.make_async_copy(tbl_sh.at[ibufs[s]], cbufs[s],
                                  gsems[s]).wait()

        def depair(s):
            cb = cbufs[s]
            ob = obufs[s]

            def row(r, _):
                pr = r // 2
                half = lax.rem(r, 2) * D
                for j in range(D // L):
                    ob[r, pl.ds(j * L, L)] = cb[pr, pl.ds(half + j * L, L)]
                return _

            lax.fori_loop(0, CH, row, 0)

        def _out_slice(m, q):
            return out_hbm.at[b0 + m, pl.ds(q * CH, CH)]

        def issue_out(m, q, s):
            pltpu.async_copy(obufs[s], _out_slice(m, q), osems[s])

        def wait_out(m, q, s):
            pltpu.make_async_copy(obufs[s], _out_slice(m, q), osems[s]).wait()

        def chunk_step(m, q, mb, first=False, last=False):
            s = q % 2
            if not (first and q < 2):
                pq, pm = (q - 2, m) if q >= 2 else (q + CPM - 2, m - 1)
                wait_out(pm, pq, s)
            wait_gather(s)
            depair(s)
            if q == 6 and not last:
                wait_maze(m + 1, 1 - mb)
                if not last:
                    pass
            if not (last and q >= 6):
                if q < 6:
                    build_and_gather(m, q + 2, s, mb)
                else:
                    build_and_gather(m + 1, q - 6, s, 1 - mb)
            issue_out(m, q, s)

        issue_maze(0, 0)
        issue_maze(1, 1)
        wait_maze(0, 0)
        build_and_gather(0, 0, 0, 0)
        build_and_gather(0, 1, 1, 0)

        def maze_pair(mi, first=False, last=False):
            for mb in range(2):
                m = mi * 2 + mb
                for q in range(CPM):
                    if q == 6 and not (last and mb == 1):
                        if not (last and mb == 0):
                            issue_maze(m + 2, mb)
                    chunk_step(m, q, mb,
                               first=(first and mb == 0),
                               last=(last and mb == 1))

        maze_pair(0, first=True)

        def group(mi, _):
            maze_pair(mi)
            return _

        lax.fori_loop(1, nb // 2 - 1, group, 0)
        maze_pair(nb // 2 - 1, last=True)

        for q in range(CPM - 2, CPM):
            wait_out(nb - 1, q, q % 2)

    return k(maze_grid, cellcat, pos128)


def kernel(maze_grid, cell_table, pos_table):
    batch, h, w = maze_grid.shape
    cellcat = jnp.concatenate(
        [jnp.repeat(cell_table, V, axis=0),
         jnp.tile(cell_table, (V, 1))], axis=1)
    pos128 = pos_table.reshape(P // 2, 2 * D)
    return _sc_encode(maze_grid.astype(jnp.int32), cellcat, pos128, batch)


# speedup vs baseline: 1.0350x; 1.0350x over previous
import functools

import jax
import jax.numpy as jnp
from jax import lax
from jax.experimental import pallas as pl
from jax.experimental.pallas import tpu as pltpu
from jax.experimental.pallas import tpu_sc as plsc

MAZE = 32
P = MAZE * MAZE
D = 64
V = 4
PP = P // 2
TBL = V * V * PP
NC, NS, L = 2, 16, 16
NW = NC * NS
CH = 128
CPM = P // CH
CHP = CH // 2


def _sc_encode(maze_grid, cellcat, pospaired, batch):
    nb = batch // NW
    rows_per_sub = TBL // NS

    mesh = plsc.VectorSubcoreMesh(core_axis_name="c", subcore_axis_name="s")

    @functools.partial(
        pl.kernel,
        out_type=jax.ShapeDtypeStruct((batch, P, D), jnp.float32),
        mesh=mesh,
        scratch_types=[
            pltpu.VMEM_SHARED((TBL, 2 * D), jnp.float32),
            pltpu.VMEM((2 * D,), jnp.float32),
        ] + [pltpu.VMEM((MAZE, MAZE), jnp.int32) for _ in range(2)]
          + [pltpu.VMEM((CHP,), jnp.int32) for _ in range(2)]
          + [pltpu.VMEM((CHP, 2 * D), jnp.float32) for _ in range(2)]
          + [pltpu.VMEM((CH, D), jnp.float32) for _ in range(2)]
          + [pltpu.SemaphoreType.DMA for _ in range(6)],
    )
    def k(maze_hbm, cell_hbm, pos_hbm, out_hbm, tbl_sh, ccbuf, *ring):
        mbufs = ring[0:2]
        ibufs = ring[2:4]
        cbufs = ring[4:6]
        obufs = ring[6:8]
        msems = ring[8:10]
        gsems = ring[10:12]
        osems = ring[12:14]
        cid = lax.axis_index("c")
        sid = lax.axis_index("s")
        wid = sid * NC + cid
        b0 = wid * nb
        lanes = lax.iota(jnp.int32, L)

        pltpu.sync_copy(cell_hbm.at[sid], ccbuf)
        ccs = [ccbuf[pl.ds(j * L, L)] for j in range(2 * D // L)]
        row0 = sid * rows_per_sub
        for kk in range(rows_per_sub // CHP):
            bb = cbufs[kk % 2]
            pltpu.sync_copy(pos_hbm.at[pl.ds(kk * CHP, CHP)], bb)

            def add_row(r, _, bb=bb):
                for j in range(2 * D // L):
                    bb[r, pl.ds(j * L, L)] += ccs[j]
                return _

            lax.fori_loop(0, CHP, add_row, 0)
            pltpu.sync_copy(bb, tbl_sh.at[pl.ds(row0 + kk * CHP, CHP)])
        plsc.subcore_barrier()

        def issue_maze(m, mb):
            pltpu.async_copy(maze_hbm.at[b0 + m], mbufs[mb], msems[mb])

        def wait_maze(m, mb):
            pltpu.make_async_copy(maze_hbm.at[b0 + m], mbufs[mb],
                                  msems[mb]).wait()

        def build_and_gather(q, s, mb):
            mbuf = mbufs[mb]
            for jj in range(CH // MAZE):
                e = mbuf[q * (CH // MAZE) + jj, pl.ds(0, L)]
                o = mbuf[q * (CH // MAZE) + jj, pl.ds(L, L)]
                pid = q * CHP + jj * L
                ibufs[s][pl.ds(jj * L, L)] = (e * V + o) * PP + pid + lanes
            pltpu.async_copy(tbl_sh.at[ibufs[s]], cbufs[s], gsems[s])

        def wait_gather(s):
            pltpu.make_async_copy(tbl_sh.at[ibufs[s]], cbufs[s],
                                  gsems[s]).wait()

        def depair(s):
            cb = cbufs[s]
            ob = obufs[s]

            def row(pr, _):
                r1 = 2 * L * (pr // L) + lax.rem(pr, L)
                for j in range(D // L):
                    ob[r1, pl.ds(j * L, L)] = cb[pr, pl.ds(j * L, L)]
                for j in range(D // L):
                    ob[r1 + L, pl.ds(j * L, L)] = cb[pr, pl.ds(D + j * L, L)]
                return _

            lax.fori_loop(0, CHP, row, 0)

        def _out_slice(m, q):
            return out_hbm.at[b0 + m, pl.ds(q * CH, CH)]

        def issue_out(m, q, s):
            pltpu.async_copy(obufs[s], _out_slice(m, q), osems[s])

        def wait_out(m, q, s):
            pltpu.make_async_copy(obufs[s], _out_slice(m, q), osems[s]).wait()

        def chunk_step(m, q, mb, first=False, last=False):
            s = q % 2
            if not (first and q < 2):
                pq, pm = (q - 2, m) if q >= 2 else (q + CPM - 2, m - 1)
                wait_out(pm, pq, s)
            wait_gather(s)
            depair(s)
            if q == 6 and not last:
                wait_maze(m + 1, 1 - mb)
            if not (last and q >= 6):
                if q < 6:
                    build_and_gather(q + 2, s, mb)
                else:
                    build_and_gather(q - 6, s, 1 - mb)
            issue_out(m, q, s)

        issue_maze(0, 0)
        issue_maze(1, 1)
        wait_maze(0, 0)
        build_and_gather(0, 0, 0)
        build_and_gather(1, 1, 0)

        def maze_pair(mi, first=False, last=False):
            for mb in range(2):
                m = mi * 2 + mb
                for q in range(CPM):
                    if q == 6 and not (last and mb == 1):
                        if not (last and mb == 0):
                            issue_maze(m + 2, mb)
                    chunk_step(m, q, mb,
                               first=(first and mb == 0),
                               last=(last and mb == 1))

        maze_pair(0, first=True)

        def group(mi, _):
            maze_pair(mi)
            return _

        lax.fori_loop(1, nb // 2 - 1, group, 0)
        maze_pair(nb // 2 - 1, last=True)

        for q in range(CPM - 2, CPM):
            wait_out(nb - 1, q, q % 2)

    return k(maze_grid, cellcat, pospaired)


def kernel(maze_grid, cell_table, pos_table):
    batch, h, w = maze_grid.shape
    cellcat = jnp.concatenate(
        [jnp.repeat(cell_table, V, axis=0),
         jnp.tile(cell_table, (V, 1))], axis=1)
    pr4 = pos_table.reshape(MAZE, 2, L, D)
    pospaired = jnp.concatenate([pr4[:, 0], pr4[:, 1]], axis=-1)
    pospaired = pospaired.reshape(PP, 2 * D)
    return _sc_encode(maze_grid.astype(jnp.int32), cellcat, pospaired, batch)
